# Initial kernel scaffold; baseline (speedup 1.0000x reference)
import jax
import jax.numpy as jnp
from jax.experimental import pallas as pl


def kernel(x, table):
    raise NotImplementedError("write your pallas kernel here")


# speedup vs baseline: 4.0853x; 4.0853x over previous
import functools

import jax
import jax.numpy as jnp
from jax import lax
from jax.experimental import pallas as pl
from jax.experimental.pallas import tpu as pltpu
from jax.experimental.pallas import tpu_sc as plsc

_NC = 2
_NS = 16
_NW = _NC * _NS
_CHUNK = 128


def _embed_lookup(idx3d, table, n_chunks, d):
    per_w = n_chunks // _NW
    n_rows = n_chunks * _CHUNK
    mesh = plsc.VectorSubcoreMesh(core_axis_name="c", subcore_axis_name="s")

    @functools.partial(
        pl.kernel,
        mesh=mesh,
        compiler_params=pltpu.CompilerParams(use_tc_tiling_on_sc=False),
        out_type=jax.ShapeDtypeStruct((n_rows, d), jnp.float32),
        scratch_types=[
            pltpu.VMEM((per_w, _CHUNK), jnp.int32),
            pltpu.VMEM((_CHUNK, d), jnp.float32),
            pltpu.SemaphoreType.DMA,
        ],
    )
    def run(x_hbm, table_hbm, out_hbm, idx_v, rows_v, gsem):
        wid = lax.axis_index("s") * _NC + lax.axis_index("c")
        row0 = wid * per_w
        pltpu.sync_copy(x_hbm.at[wid], idx_v)

        def chunk(j, carry):
            pltpu.async_copy(table_hbm.at[idx_v.at[j]], rows_v, gsem).wait()
            base = pl.multiple_of((row0 + j) * _CHUNK, _CHUNK)
            pltpu.sync_copy(rows_v, out_hbm.at[pl.ds(base, _CHUNK)])
            return carry

        lax.fori_loop(0, per_w, chunk, 0)

    return run(idx3d, table)


def kernel(x, table):
    b, h = x.shape
    v, d = table.shape
    n = b * h
    n_chunks = n // _CHUNK
    idx3d = x.reshape(_NW, n_chunks // _NW, _CHUNK).astype(jnp.int32)
    out = _embed_lookup(idx3d, table, n_chunks, d)
    return out.reshape(b, h, d)


# speedup vs baseline: 4.6621x; 1.1412x over previous
import functools

import jax
import jax.numpy as jnp
from jax import lax
from jax.experimental import pallas as pl
from jax.experimental.pallas import tpu as pltpu
from jax.experimental.pallas import tpu_sc as plsc

_NC = 2
_NS = 16
_NW = _NC * _NS
_CHUNK = 128
_G = 2
_NB = 5
_L = 3


def _embed_lookup(idx3d, table, n_chunks, d):
    per_w = n_chunks // _NW
    groups = per_w // _G
    n_rows = n_chunks * _CHUNK
    mesh = plsc.VectorSubcoreMesh(core_axis_name="c", subcore_axis_name="s")

    @functools.partial(
        pl.kernel,
        mesh=mesh,
        compiler_params=pltpu.CompilerParams(use_tc_tiling_on_sc=False),
        out_type=jax.ShapeDtypeStruct((n_rows, d), jnp.float32),
        scratch_types=[
            pltpu.VMEM((per_w, _CHUNK), jnp.int32),
            pltpu.VMEM((_NB, _G * _CHUNK, d), jnp.float32),
            pltpu.SemaphoreType.DMA((_NB,)),
            pltpu.SemaphoreType.DMA((_NB,)),
        ],
    )
    def run(x_hbm, table_hbm, out_hbm, idx_v, bufs, gsem, osem):
        wid = lax.axis_index("s") * _NC + lax.axis_index("c")
        row0 = wid * per_w
        pltpu.sync_copy(x_hbm.at[wid], idx_v)

        def g_desc(g, b, i):
            return pltpu.make_async_copy(
                table_hbm.at[idx_v.at[g * _G + i]],
                bufs.at[b, pl.ds(i * _CHUNK, _CHUNK)],
                gsem.at[b],
            )

        def o_desc(g, b):
            base = pl.multiple_of((row0 + g * _G) * _CHUNK, _G * _CHUNK)
            return pltpu.make_async_copy(
                bufs.at[b],
                out_hbm.at[pl.ds(base, _G * _CHUNK)],
                osem.at[b],
            )

        for g in range(_L):
            for i in range(_G):
                g_desc(g, g % _NB, i).start()

        def outer(o, carry):
            for p in range(_NB):
                j = o * _NB + p
                gf = j + _L
                bf = (p + _L) % _NB

                @pl.when(gf < groups)
                def _fire():
                    @pl.when(gf >= _NB)
                    def _reuse():
                        o_desc(gf - _NB, bf).wait()

                    for i in range(_G):
                        g_desc(gf, bf, i).start()

                for i in range(_G):
                    g_desc(j, p, i).wait()
                o_desc(j, p).start()
            return carry

        lax.fori_loop(0, groups // _NB, outer, 0)

        for b in range(_NB):
            o_desc(groups - _NB + b, b).wait()

    return run(idx3d, table)


def kernel(x, table):
    b, h = x.shape
    v, d = table.shape
    n = b * h
    n_chunks = n // _CHUNK
    idx3d = x.reshape(_NW, n_chunks // _NW, _CHUNK).astype(jnp.int32)
    out = _embed_lookup(idx3d, table, n_chunks, d)
    return out.reshape(b, h, d)


# speedup vs baseline: 4.6635x; 1.0003x over previous
import functools

import jax
import jax.numpy as jnp
from jax import lax
from jax.experimental import pallas as pl
from jax.experimental.pallas import tpu as pltpu
from jax.experimental.pallas import tpu_sc as plsc

_NC = 2
_NS = 16
_NW = _NC * _NS
_GB = 8
_NB = 4
_L = 2


def _embed_lookup(x, table):
    b, h = x.shape
    _, d = table.shape
    per_w = b // _NW
    groups = per_w // _GB
    mesh = plsc.VectorSubcoreMesh(core_axis_name="c", subcore_axis_name="s")

    @functools.partial(
        pl.kernel,
        mesh=mesh,
        compiler_params=pltpu.CompilerParams(use_tc_tiling_on_sc=False),
        out_type=jax.ShapeDtypeStruct((b, h, d), jnp.float32),
        scratch_types=[
            pltpu.VMEM((per_w, h), jnp.int32),
            pltpu.VMEM((_NB, _GB, h, d), jnp.float32),
            pltpu.SemaphoreType.DMA((_NB,)),
            pltpu.SemaphoreType.DMA((_NB,)),
        ],
    )
    def run(x_hbm, table_hbm, out_hbm, idx_v, bufs, gsem, osem):
        wid = lax.axis_index("s") * _NC + lax.axis_index("c")
        batch0 = wid * per_w
        pltpu.sync_copy(x_hbm.at[pl.ds(batch0, per_w)], idx_v)

        def g_desc(g, rb, i):
            return pltpu.make_async_copy(
                table_hbm.at[idx_v.at[g * _GB + i]],
                bufs.at[rb, i],
                gsem.at[rb],
            )

        def o_desc(g, rb):
            base = pl.multiple_of(batch0 + g * _GB, _GB)
            return pltpu.make_async_copy(
                bufs.at[rb],
                out_hbm.at[pl.ds(base, _GB)],
                osem.at[rb],
            )

        for g in range(_L):
            for i in range(_GB):
                g_desc(g, g % _NB, i).start()

        def outer(o, carry):
            for p in range(_NB):
                j = o * _NB + p
                gf = j + _L
                bf = (p + _L) % _NB

                @pl.when(gf < groups)
                def _fire():
                    @pl.when(gf >= _NB)
                    def _reuse():
                        o_desc(gf - _NB, bf).wait()

                    for i in range(_GB):
                        g_desc(gf, bf, i).start()

                for i in range(_GB):
                    g_desc(j, p, i).wait()
                o_desc(j, p).start()
            return carry

        lax.fori_loop(0, groups // _NB, outer, 0)

        for rb in range(_NB):
            o_desc(groups - _NB + rb, rb).wait()

    return run(x, table)


def kernel(x, table):
    return _embed_lookup(x.astype(jnp.int32), table)
